# baseline (speedup 1.0000x reference)
import functools

import jax
import jax.numpy as jnp
from jax import lax
from jax.experimental import pallas as pl
from jax.experimental.pallas import tpu as pltpu
from jax.experimental.pallas import tpu_sc as plsc

NQ = 10000
NS = 10000
D = 128
E = 320000
B = 1024
N = 200

CH = 128
EPC = 2560
EP = EPC * CH
ECT = EPC // 32
R = B * N
RPW = R // 32
OCT = RPW // CH
NQP = 10240
RPT = NQP // 16

_mesh = plsc.VectorSubcoreMesh(core_axis_name="c", subcore_axis_name="s",
                               num_cores=2)


@functools.partial(
    pl.kernel,
    out_type=(jax.ShapeDtypeStruct((2 * NQP, D), jnp.float32),
              jax.ShapeDtypeStruct((2 * NQP,), jnp.float32)),
    mesh=_mesh,
    scratch_types=[
        pltpu.VMEM((CH, D), jnp.float32),
        pltpu.VMEM((CH, D), jnp.float32),
        pltpu.VMEM((8, CH), jnp.int32),
        pltpu.VMEM((8, CH), jnp.int32),
        pltpu.VMEM((8, CH), jnp.int32),
        pltpu.VMEM((8, CH), jnp.int32),
        pltpu.VMEM((CH,), jnp.float32),
        pltpu.VMEM_SHARED((NQP, D), jnp.float32),
        pltpu.VMEM_SHARED((NQP,), jnp.float32),
        pltpu.SemaphoreType.DMA,
        pltpu.SemaphoreType.DMA,
        pltpu.SemaphoreType.DMA,
        pltpu.SemaphoreType.DMA,
        pltpu.SemaphoreType.DMA,
        pltpu.SemaphoreType.DMA,
    ],
)
def _edge_sweep(ridx_hbm, cidx_hbm, s_hbm, zacc_hbm, zdeg_hbm,
                pacc_hbm, pdeg_hbm,
                rba, rbb, colb0, rowb0, colb1, rowb1, ones,
                acc_sp, deg_sp, sema, semb, semsa, semsb, semd, semi):
    ci = lax.axis_index("c")
    s = lax.axis_index("s")

    pltpu.sync_copy(zacc_hbm, acc_sp.at[pl.ds(s * RPT, RPT)])
    pltpu.sync_copy(zdeg_hbm, deg_sp.at[pl.ds(s * RPT, RPT)])
    for k in range(CH // 16):
        ones[pl.ds(k * 16, 16)] = jnp.ones((16,), jnp.float32)
    plsc.subcore_barrier()

    bufs = (rba, rbb)
    gsems = (sema, semb)
    ssems = (semsa, semsb)
    base0 = ci * (EPC // 2) + s * ECT

    def run_macro(colX, rowX):
        gd = [None] * 8
        sd = [None] * 8
        dd = [None] * 8
        gd[0] = pltpu.async_copy(s_hbm.at[colX.at[0]], rba, sema)
        for k in range(8):
            cur = bufs[k % 2]
            gd[k].wait()
            if k < 7:
                if k >= 1:
                    sd[k - 1].wait()
                gd[k + 1] = pltpu.async_copy(
                    s_hbm.at[colX.at[k + 1]], bufs[(k + 1) % 2],
                    gsems[(k + 1) % 2])
            sd[k] = pltpu.async_copy(cur, acc_sp.at[rowX.at[k]],
                                     ssems[k % 2], add=True)
            dd[k] = pltpu.async_copy(ones, deg_sp.at[rowX.at[k]],
                                     semd, add=True)
        sd[6].wait()
        sd[7].wait()
        for k in range(8):
            dd[k].wait()

    def wait_idx(colX, rowX):
        pltpu.make_async_copy(cidx_hbm.at[pl.ds(0, 8)], colX, semi).wait()
        pltpu.make_async_copy(ridx_hbm.at[pl.ds(0, 8)], rowX, semi).wait()

    def fire_idx(mb, colX, rowX):
        pltpu.async_copy(cidx_hbm.at[pl.ds(mb, 8)], colX, semi)
        pltpu.async_copy(ridx_hbm.at[pl.ds(mb, 8)], rowX, semi)

    fire_idx(base0, colb0, rowb0)

    def pair(t, _):
        m1 = base0 + (2 * t + 1) * 8
        m2 = base0 + jnp.minimum((2 * t + 2) * 8, (ECT - 8))
        wait_idx(colb0, rowb0)
        fire_idx(m1, colb1, rowb1)
        run_macro(colb0, rowb0)
        wait_idx(colb1, rowb1)
        fire_idx(m2, colb0, rowb0)
        run_macro(colb1, rowb1)
        return 0
    lax.fori_loop(0, ECT // 16, pair, 0)
    wait_idx(colb0, rowb0)
    plsc.subcore_barrier()

    pltpu.sync_copy(acc_sp.at[pl.ds(s * RPT, RPT)],
                    pacc_hbm.at[pl.ds(ci * NQP + s * RPT, RPT)])
    pltpu.sync_copy(deg_sp.at[pl.ds(s * RPT, RPT)],
                    pdeg_hbm.at[pl.ds(ci * NQP + s * RPT, RPT)])


def _combine_body(p0_ref, p1_ref, d0_ref, d1_ref, q_ref):
    d = d0_ref[...] + d1_ref[...]
    rd = 1.0 / jnp.maximum(d, 1.0)
    q_ref[...] = (p0_ref[...] + p1_ref[...]) * rd


_NB = NQP // 1024

_combine = pl.pallas_call(
    _combine_body,
    grid=(_NB,),
    in_specs=[
        pl.BlockSpec((1024, D), lambda g: (g, 0)),
        pl.BlockSpec((1024, D), lambda g: (g + _NB, 0)),
        pl.BlockSpec((1024, 1), lambda g: (g, 0)),
        pl.BlockSpec((1024, 1), lambda g: (g + _NB, 0)),
    ],
    out_specs=pl.BlockSpec((1024, D), lambda g: (g, 0)),
    out_shape=jax.ShapeDtypeStruct((NQP, D), jnp.float32),
)


@functools.partial(
    pl.kernel,
    out_type=jax.ShapeDtypeStruct((R, D), jnp.float32),
    mesh=_mesh,
    scratch_types=[
        pltpu.VMEM((RPW,), jnp.int32),
        pltpu.VMEM((CH, D), jnp.float32),
        pltpu.VMEM((CH, D), jnp.float32),
        pltpu.SemaphoreType.DMA,
        pltpu.SemaphoreType.DMA,
    ],
)
def _lookup(x_hbm, q_hbm, out_hbm, xb, rba, rbb, sema, semb):
    ci = lax.axis_index("c")
    s = lax.axis_index("s")
    w = ci * 16 + s

    pltpu.sync_copy(x_hbm.at[pl.ds(w * RPW, RPW)], xb)

    pltpu.async_copy(q_hbm.at[xb.at[pl.ds(0, CH)]], rba, sema)

    def pair(t, _):
        j0 = 2 * t
        pltpu.make_async_copy(q_hbm.at[xb.at[pl.ds(0, CH)]], rba, sema).wait()
        pltpu.async_copy(q_hbm.at[xb.at[pl.ds((j0 + 1) * CH, CH)]], rbb, semb)
        pltpu.sync_copy(rba, out_hbm.at[pl.ds(w * RPW + j0 * CH, CH)])
        j2 = jnp.minimum(j0 + 2, OCT - 2)
        pltpu.make_async_copy(q_hbm.at[xb.at[pl.ds(0, CH)]], rbb, semb).wait()
        pltpu.async_copy(q_hbm.at[xb.at[pl.ds(j2 * CH, CH)]], rba, sema)
        pltpu.sync_copy(rbb, out_hbm.at[pl.ds(w * RPW + (j0 + 1) * CH, CH)])
        return 0
    lax.fori_loop(0, OCT // 2, pair, 0)
    pltpu.make_async_copy(q_hbm.at[xb.at[pl.ds(0, CH)]], rba, sema).wait()


def kernel(x, a_rows, a_cols, a_vals, q_table, s_table):
    del a_vals, q_table
    pad = EP - E
    pad_rows = NQ + (jnp.arange(pad, dtype=jnp.int32) % 16)
    pad_cols = (jnp.arange(pad, dtype=jnp.int32) * 131) % NS
    ridx2d = jnp.concatenate([a_rows, pad_rows]).reshape(EPC, CH)
    cidx2d = jnp.concatenate([a_cols, pad_cols]).reshape(EPC, CH)
    zacc = jnp.zeros((RPT, D), jnp.float32)
    zdeg = jnp.zeros((RPT,), jnp.float32)
    pacc, pdeg = _edge_sweep(ridx2d, cidx2d, s_table, zacc, zdeg)
    pdeg_col = pdeg.reshape(2 * NQP, 1)
    q_full = _combine(pacc, pacc, pdeg_col, pdeg_col)
    out = _lookup(x.reshape(R), q_full)
    return out.reshape(B, N, D)


# speedup vs baseline: 1.1954x; 1.1954x over previous
import functools

import jax
import jax.numpy as jnp
from jax import lax
from jax.experimental import pallas as pl
from jax.experimental.pallas import tpu as pltpu
from jax.experimental.pallas import tpu_sc as plsc

NQ = 10000
NS = 10000
D = 128
E = 320000
B = 1024
N = 200

CH = 128
EPC = 2560
EP = EPC * CH
ECT = EPC // 32
R = B * N
RPW = R // 32
OCT = RPW // CH
NQP = 10240
RPT = NQP // 16

_mesh = plsc.VectorSubcoreMesh(core_axis_name="c", subcore_axis_name="s",
                               num_cores=2)


@functools.partial(
    pl.kernel,
    out_type=(jax.ShapeDtypeStruct((2 * NQP, D), jnp.float32),
              jax.ShapeDtypeStruct((2 * NQP,), jnp.float32)),
    mesh=_mesh,
    scratch_types=[
        pltpu.VMEM((CH, D), jnp.float32),
        pltpu.VMEM((CH, D), jnp.float32),
        pltpu.VMEM((8, CH), jnp.int32),
        pltpu.VMEM((8, CH), jnp.int32),
        pltpu.VMEM((8, CH), jnp.int32),
        pltpu.VMEM((8, CH), jnp.int32),
        pltpu.VMEM((CH,), jnp.float32),
        pltpu.VMEM_SHARED((NQP, D), jnp.float32),
        pltpu.VMEM_SHARED((NQP,), jnp.float32),
        pltpu.SemaphoreType.DMA,
        pltpu.SemaphoreType.DMA,
        pltpu.SemaphoreType.DMA,
        pltpu.SemaphoreType.DMA,
        pltpu.SemaphoreType.DMA,
        pltpu.SemaphoreType.DMA,
    ],
)
def _edge_sweep(ridx_hbm, cidx_hbm, s_hbm, zacc_hbm, zdeg_hbm,
                pacc_hbm, pdeg_hbm,
                rba, rbb, colb0, rowb0, colb1, rowb1, ones,
                acc_sp, deg_sp, sema, semb, semsa, semsb, semd, semi):
    ci = lax.axis_index("c")
    s = lax.axis_index("s")

    pltpu.sync_copy(zacc_hbm, acc_sp.at[pl.ds(s * RPT, RPT)])
    pltpu.sync_copy(zdeg_hbm, deg_sp.at[pl.ds(s * RPT, RPT)])
    for k in range(CH // 16):
        ones[pl.ds(k * 16, 16)] = jnp.ones((16,), jnp.float32)
    plsc.subcore_barrier()

    bufs = (rba, rbb)
    gsems = (sema, semb)
    ssems = (semsa, semsb)
    base0 = ci * (EPC // 2) + s * ECT

    def run_macro(colX, rowX):
        gd = [None] * 8
        sd = [None] * 8
        dd = [None] * 8
        gd[0] = pltpu.async_copy(s_hbm.at[colX.at[0]], rba, sema)
        for k in range(8):
            cur = bufs[k % 2]
            gd[k].wait()
            if k < 7:
                if k >= 1:
                    sd[k - 1].wait()
                gd[k + 1] = pltpu.async_copy(
                    s_hbm.at[colX.at[k + 1]], bufs[(k + 1) % 2],
                    gsems[(k + 1) % 2])
            sd[k] = pltpu.async_copy(cur, acc_sp.at[rowX.at[k]],
                                     ssems[k % 2], add=True)
            dd[k] = pltpu.async_copy(ones, deg_sp.at[rowX.at[k]],
                                     semd, add=True)
        sd[6].wait()
        sd[7].wait()
        for k in range(8):
            dd[k].wait()

    def wait_idx(colX, rowX):
        pltpu.make_async_copy(cidx_hbm.at[pl.ds(0, 8)], colX, semi).wait()
        pltpu.make_async_copy(ridx_hbm.at[pl.ds(0, 8)], rowX, semi).wait()

    def fire_idx(mb, colX, rowX):
        pltpu.async_copy(cidx_hbm.at[pl.ds(mb, 8)], colX, semi)
        pltpu.async_copy(ridx_hbm.at[pl.ds(mb, 8)], rowX, semi)

    fire_idx(base0, colb0, rowb0)

    def pair(t, _):
        m1 = base0 + (2 * t + 1) * 8
        m2 = base0 + jnp.minimum((2 * t + 2) * 8, (ECT - 8))
        wait_idx(colb0, rowb0)
        fire_idx(m1, colb1, rowb1)
        run_macro(colb0, rowb0)
        wait_idx(colb1, rowb1)
        fire_idx(m2, colb0, rowb0)
        run_macro(colb1, rowb1)
        return 0
    lax.fori_loop(0, ECT // 16, pair, 0)
    wait_idx(colb0, rowb0)
    plsc.subcore_barrier()

    pltpu.sync_copy(acc_sp.at[pl.ds(s * RPT, RPT)],
                    pacc_hbm.at[pl.ds(ci * NQP + s * RPT, RPT)])
    pltpu.sync_copy(deg_sp.at[pl.ds(s * RPT, RPT)],
                    pdeg_hbm.at[pl.ds(ci * NQP + s * RPT, RPT)])


def _combine_body(p0_ref, p1_ref, d0_ref, d1_ref, q_ref):
    d = d0_ref[...] + d1_ref[...]
    rd = 1.0 / jnp.maximum(d, 1.0)
    q_ref[...] = (p0_ref[...] + p1_ref[...]) * rd


_NB = NQP // 1024

_combine = pl.pallas_call(
    _combine_body,
    grid=(_NB,),
    in_specs=[
        pl.BlockSpec((1024, D), lambda g: (g, 0)),
        pl.BlockSpec((1024, D), lambda g: (g + _NB, 0)),
        pl.BlockSpec((1024, 1), lambda g: (g, 0)),
        pl.BlockSpec((1024, 1), lambda g: (g + _NB, 0)),
    ],
    out_specs=pl.BlockSpec((1024, D), lambda g: (g, 0)),
    out_shape=jax.ShapeDtypeStruct((NQP, D), jnp.float32),
)


@functools.partial(
    pl.kernel,
    out_type=jax.ShapeDtypeStruct((R, D), jnp.float32),
    mesh=_mesh,
    scratch_types=[
        pltpu.VMEM((RPW,), jnp.int32),
        pltpu.VMEM((CH, D), jnp.float32),
        pltpu.VMEM((CH, D), jnp.float32),
        pltpu.VMEM_SHARED((NQP, D), jnp.float32),
        pltpu.SemaphoreType.DMA,
        pltpu.SemaphoreType.DMA,
    ],
)
def _lookup(x_hbm, q_hbm, out_hbm, xb, rba, rbb, q_sp, sema, semb):
    ci = lax.axis_index("c")
    s = lax.axis_index("s")
    w = ci * 16 + s

    pltpu.sync_copy(q_hbm.at[pl.ds(s * RPT, RPT)], q_sp.at[pl.ds(s * RPT, RPT)])
    pltpu.sync_copy(x_hbm.at[pl.ds(w * RPW, RPW)], xb)
    plsc.subcore_barrier()

    pltpu.async_copy(q_sp.at[xb.at[pl.ds(0, CH)]], rba, sema)

    def pair(t, _):
        j0 = 2 * t
        pltpu.make_async_copy(q_sp.at[xb.at[pl.ds(0, CH)]], rba, sema).wait()
        pltpu.async_copy(q_sp.at[xb.at[pl.ds((j0 + 1) * CH, CH)]], rbb, semb)
        pltpu.sync_copy(rba, out_hbm.at[pl.ds(w * RPW + j0 * CH, CH)])
        j2 = jnp.minimum(j0 + 2, OCT - 2)
        pltpu.make_async_copy(q_sp.at[xb.at[pl.ds(0, CH)]], rbb, semb).wait()
        pltpu.async_copy(q_sp.at[xb.at[pl.ds(j2 * CH, CH)]], rba, sema)
        pltpu.sync_copy(rbb, out_hbm.at[pl.ds(w * RPW + (j0 + 1) * CH, CH)])
        return 0
    lax.fori_loop(0, OCT // 2, pair, 0)
    pltpu.make_async_copy(q_sp.at[xb.at[pl.ds(0, CH)]], rba, sema).wait()


def kernel(x, a_rows, a_cols, a_vals, q_table, s_table):
    del a_vals, q_table
    pad = EP - E
    pad_rows = NQ + (jnp.arange(pad, dtype=jnp.int32) % 16)
    pad_cols = (jnp.arange(pad, dtype=jnp.int32) * 131) % NS
    ridx2d = jnp.concatenate([a_rows, pad_rows]).reshape(EPC, CH)
    cidx2d = jnp.concatenate([a_cols, pad_cols]).reshape(EPC, CH)
    zacc = jnp.zeros((RPT, D), jnp.float32)
    zdeg = jnp.zeros((RPT,), jnp.float32)
    pacc, pdeg = _edge_sweep(ridx2d, cidx2d, s_table, zacc, zdeg)
    pdeg_col = pdeg.reshape(2 * NQP, 1)
    q_full = _combine(pacc, pacc, pdeg_col, pdeg_col)
    out = _lookup(x.reshape(R), q_full)
    return out.reshape(B, N, D)
